# speedup vs baseline: 13.7703x; 13.7703x over previous
import functools

import jax
import jax.numpy as jnp
from jax import lax
from jax.experimental import pallas as pl
from jax.experimental.pallas import tpu as pltpu
from jax.experimental.pallas import tpu_sc as plsc

_INFO = plsc.get_sparse_core_info()
_NC = _INFO.num_cores
_NS = _INFO.num_subcores
_NW = _NC * _NS
_LANES = _INFO.num_lanes


def _make_sc_kernel(V, D, B, L):
    assert L % 2 == 0
    half = L // 2
    assert B % (8 * _NW) == 0 and D % _LANES == 0
    bpw = B // _NW
    ncol = D // _LANES

    mesh = plsc.VectorSubcoreMesh(core_axis_name="c", subcore_axis_name="s")

    @functools.partial(
        pl.kernel,
        mesh=mesh,
        out_type=jax.ShapeDtypeStruct((B, D), jnp.float32),
        scratch_types=[
            pltpu.VMEM((bpw, 2, half), jnp.int32),
            pltpu.VMEM((2, half, D), jnp.float32),
            pltpu.VMEM((2, half, D), jnp.float32),
            pltpu.VMEM((bpw, D), jnp.float32),
            pltpu.SemaphoreType.DMA,
            pltpu.SemaphoreType.DMA,
        ],
    )
    def k(table_hbm, idx_hbm, out_hbm, idx_v, rows0, rows1, out_v, sem0, sem1):
        wid = lax.axis_index("s") * _NC + lax.axis_index("c")
        base = wid * bpw
        pltpu.sync_copy(idx_hbm.at[pl.ds(base, bpw)], idx_v)

        def start(i, rows, sem):
            for h in range(2):
                pltpu.async_copy(table_hbm.at[idx_v.at[i, h]], rows.at[h], sem)

        def wait(i, rows, sem):
            for h in range(2):
                pltpu.make_async_copy(
                    table_hbm.at[idx_v.at[i, h]], rows.at[h], sem
                ).wait()

        def compute(i, rows):
            def row_body(j, acc):
                return tuple(
                    jnp.maximum(
                        jnp.maximum(acc[c], rows[0, j, pl.ds(c * _LANES, _LANES)]),
                        rows[1, j, pl.ds(c * _LANES, _LANES)],
                    )
                    for c in range(ncol)
                )

            acc0 = tuple(
                jnp.maximum(
                    rows[0, 0, pl.ds(c * _LANES, _LANES)],
                    rows[1, 0, pl.ds(c * _LANES, _LANES)],
                )
                for c in range(ncol)
            )
            acc = lax.fori_loop(1, half, row_body, acc0)
            for c in range(ncol):
                out_v[i, pl.ds(c * _LANES, _LANES)] = acc[c]

        start(0, rows0, sem0)
        start(1, rows1, sem1)

        def loop_body(step, _):
            bufs = ((rows0, sem0), (rows1, sem1))
            i2 = step * 2
            for b in range(2):
                i = i2 + b
                rows, sem = bufs[b]
                wait(i, rows, sem)
                compute(i, rows)

                @pl.when(i + 2 < bpw)
                def _():
                    start(i + 2, rows, sem)

            return ()

        lax.fori_loop(0, bpw // 2, loop_body, ())
        pltpu.sync_copy(out_v, out_hbm.at[pl.ds(base, bpw)])

    return k


def kernel(input, table):
    B, L = input.shape
    V, D = table.shape
    idx3 = input.astype(jnp.int32).reshape(B, 2, L // 2)
    return _make_sc_kernel(V, D, B, L)(table, idx3)


# speedup vs baseline: 17.0393x; 1.2374x over previous
import functools

import jax
import jax.numpy as jnp
from jax import lax
from jax.experimental import pallas as pl
from jax.experimental.pallas import tpu as pltpu
from jax.experimental.pallas import tpu_sc as plsc

_INFO = plsc.get_sparse_core_info()
_NC = _INFO.num_cores
_NS = _INFO.num_subcores
_NW = _NC * _NS
_LANES = _INFO.num_lanes


def _make_sc_kernel(V, D, B, L):
    assert L % 2 == 0
    half = L // 2
    assert B % (8 * _NW) == 0 and D % _LANES == 0
    bpw = B // _NW
    ncol = D // _LANES

    mesh = plsc.VectorSubcoreMesh(core_axis_name="c", subcore_axis_name="s")

    @functools.partial(
        pl.kernel,
        mesh=mesh,
        out_type=jax.ShapeDtypeStruct((B, D), jnp.float32),
        scratch_types=[
            pltpu.VMEM((bpw, 2, half), jnp.int32),
            pltpu.VMEM((2, half, D), jnp.float32),
            pltpu.VMEM((2, half, D), jnp.float32),
            pltpu.VMEM((2, half, D), jnp.float32),
            pltpu.VMEM((bpw, D), jnp.float32),
            pltpu.SemaphoreType.DMA,
            pltpu.SemaphoreType.DMA,
            pltpu.SemaphoreType.DMA,
        ],
    )
    def k(table_hbm, idx_hbm, out_hbm, idx_v, rows0, rows1, rows2, out_v,
          sem0, sem1, sem2):
        wid = lax.axis_index("s") * _NC + lax.axis_index("c")
        base = wid * bpw
        pltpu.sync_copy(idx_hbm.at[pl.ds(base, bpw)], idx_v)

        def start(i, rows, sem):
            for h in range(2):
                pltpu.async_copy(table_hbm.at[idx_v.at[i, h]], rows.at[h], sem)

        def wait(i, rows, sem):
            for h in range(2):
                pltpu.make_async_copy(
                    table_hbm.at[idx_v.at[i, h]], rows.at[h], sem
                ).wait()

        def compute(i, rows):
            def row_body(j2, acc):
                j = j2 * 2
                out = []
                for c in range(ncol):
                    s = pl.ds(c * _LANES, _LANES)
                    m0 = jnp.maximum(rows[0, j, s], rows[1, j, s])
                    m1 = jnp.maximum(rows[0, j + 1, s], rows[1, j + 1, s])
                    out.append(jnp.maximum(acc[c], jnp.maximum(m0, m1)))
                return tuple(out)

            acc0 = tuple(
                jnp.full((_LANES,), -jnp.inf, jnp.float32) for _ in range(ncol)
            )
            acc = lax.fori_loop(0, half // 2, row_body, acc0)
            for c in range(ncol):
                out_v[i, pl.ds(c * _LANES, _LANES)] = acc[c]

        bufs = ((rows0, sem0), (rows1, sem1), (rows2, sem2))
        nbuf = len(bufs)
        for b in range(nbuf):
            start(b, bufs[b][0], bufs[b][1])

        def loop_body(step, _):
            i0 = step * nbuf
            for b in range(nbuf):
                i = i0 + b
                rows, sem = bufs[b]
                wait(i, rows, sem)
                compute(i, rows)

                @pl.when(i + nbuf < bpw)
                def _():
                    start(i + nbuf, rows, sem)

            return ()

        lax.fori_loop(0, bpw // nbuf, loop_body, ())
        for i in range(bpw - bpw % nbuf, bpw):
            rows, sem = bufs[i % nbuf]
            wait(i, rows, sem)
            compute(i, rows)
        pltpu.sync_copy(out_v, out_hbm.at[pl.ds(base, bpw)])

    return k


def kernel(input, table):
    B, L = input.shape
    V, D = table.shape
    idx3 = input.astype(jnp.int32).reshape(B, 2, L // 2)
    return _make_sc_kernel(V, D, B, L)(table, idx3)
